# Initial kernel scaffold; baseline (speedup 1.0000x reference)
import jax
import jax.numpy as jnp
from jax.experimental import pallas as pl


def kernel(points):
    raise NotImplementedError("write your pallas kernel here")


# speedup vs baseline: 9.3895x; 9.3895x over previous
import functools

import jax
import jax.numpy as jnp
from jax.experimental import pallas as pl
from jax.experimental.pallas import tpu as pltpu

K = 16
BQ = 256
CPAD = 8


def _knn_block(pts_nc_ref, pts_cn_ref, idx_ref, *, n):
    b = pl.program_id(0)
    q = pts_nc_ref[0]
    p = pts_cn_ref[0]
    sq_q = jnp.sum(q * q, axis=1, keepdims=True)
    sq_p = jnp.sum(p * p, axis=0, keepdims=True)
    inner = jax.lax.dot_general(
        q, p, (((1,), (0,)), ((), ())),
        preferred_element_type=jnp.float32)
    d = sq_q + sq_p - 2.0 * inner
    iota = jax.lax.broadcasted_iota(jnp.int32, (BQ, n), 1)
    cols = []
    for _ in range(K):
        m = jnp.min(d, axis=1, keepdims=True)
        cand = jnp.where(d <= m, iota, n)
        a = jnp.min(cand, axis=1, keepdims=True)
        cols.append(a)
        d = jnp.where(iota == a, jnp.float32(jnp.inf), d)
    idx_ref[0] = jnp.concatenate(cols, axis=1) + b * n


def kernel(points):
    B, N, C = points.shape
    pts_nc = jnp.pad(points, ((0, 0), (0, 0), (0, CPAD - C)))
    pts_cn = jnp.transpose(pts_nc, (0, 2, 1))
    grid = (B, N // BQ)
    dst = pl.pallas_call(
        functools.partial(_knn_block, n=N),
        grid=grid,
        in_specs=[
            pl.BlockSpec((1, BQ, CPAD), lambda b, i: (b, i, 0)),
            pl.BlockSpec((1, CPAD, N), lambda b, i: (b, 0, 0)),
        ],
        out_specs=pl.BlockSpec((1, BQ, K), lambda b, i: (b, i, 0)),
        out_shape=jax.ShapeDtypeStruct((B, N, K), jnp.int32),
    )(pts_nc, pts_cn)
    src = jnp.broadcast_to(
        jnp.arange(B * N, dtype=jnp.int32).reshape(B * N, 1), (B * N, K))
    return jnp.stack([src.reshape(-1), dst.reshape(-1)], axis=0)


# speedup vs baseline: 12.6410x; 1.3463x over previous
import functools

import jax
import jax.numpy as jnp
from jax.experimental import pallas as pl
from jax.experimental.pallas import tpu as pltpu

K = 16
BQ = 256
CPAD = 8


def _knn_block(pts_nc_ref, pts_cn_ref, idx_ref, *, n):
    b = pl.program_id(0)
    q = pts_nc_ref[0]
    p = pts_cn_ref[0]
    sq_q = jnp.sum(q * q, axis=1, keepdims=True)
    sq_p = jnp.sum(p * p, axis=0, keepdims=True)
    inner = jax.lax.dot_general(
        q, p, (((1,), (0,)), ((), ())),
        preferred_element_type=jnp.float32)
    d = sq_q + sq_p - 2.0 * inner
    iota = jax.lax.broadcasted_iota(jnp.int32, (BQ, n), 1).astype(jnp.float32)
    nf = jnp.float32(n)
    cols = []
    for _ in range(K):
        m = jnp.min(d, axis=1, keepdims=True)
        cand = jnp.where(d <= m, iota, nf)
        a = jnp.min(cand, axis=1, keepdims=True)
        cols.append(a)
        d = jnp.where(iota == a, jnp.float32(jnp.inf), d)
    idx = jnp.concatenate(cols, axis=1).astype(jnp.int32)
    idx_ref[0] = idx + b * n


def kernel(points):
    B, N, C = points.shape
    pts_nc = jnp.pad(points, ((0, 0), (0, 0), (0, CPAD - C)))
    pts_cn = jnp.transpose(pts_nc, (0, 2, 1))
    grid = (B, N // BQ)
    dst = pl.pallas_call(
        functools.partial(_knn_block, n=N),
        grid=grid,
        in_specs=[
            pl.BlockSpec((1, BQ, CPAD), lambda b, i: (b, i, 0)),
            pl.BlockSpec((1, CPAD, N), lambda b, i: (b, 0, 0)),
        ],
        out_specs=pl.BlockSpec((1, BQ, K), lambda b, i: (b, i, 0)),
        out_shape=jax.ShapeDtypeStruct((B, N, K), jnp.int32),
    )(pts_nc, pts_cn)
    src = jnp.broadcast_to(
        jnp.arange(B * N, dtype=jnp.int32).reshape(B * N, 1), (B * N, K))
    return jnp.stack([src.reshape(-1), dst.reshape(-1)], axis=0)
